# baseline (speedup 1.0000x reference)
import functools

import jax
import jax.numpy as jnp
from jax import lax
from jax.experimental import pallas as pl
from jax.experimental.pallas import tpu as pltpu
from jax.experimental.pallas import tpu_sc as plsc

_H = 256
_CG = 32
_NQ = 128
_NF = 3
_NP = 6
_K = 8
_WR = 4
_QROWS = 4
_HALO = 48


def _shift_edge(a, dy, dx):
    if dy > 0:
        a = jnp.concatenate(
            [a[dy:, :], jnp.broadcast_to(a[_CG - 1:_CG, :], (dy, _CG))], axis=0)
    elif dy < 0:
        a = jnp.concatenate(
            [jnp.broadcast_to(a[0:1, :], (-dy, _CG)), a[:dy, :]], axis=0)
    if dx > 0:
        a = jnp.concatenate(
            [a[:, dx:], jnp.broadcast_to(a[:, _CG - 1:_CG], (_CG, dx))], axis=1)
    elif dx < 0:
        a = jnp.concatenate(
            [jnp.broadcast_to(a[:, 0:1], (_CG, -dx)), a[:, :dx]], axis=1)
    return a


def _corr_body(c0_ref, c1_ref, cf_ref, cb_ref):
    for n in range(_NF):
        for dref, (a_ref, b_ref) in ((cf_ref, (c0_ref, c1_ref)),
                                     (cb_ref, (c1_ref, c0_ref))):
            v0 = [a_ref[n, ch] for ch in range(3)]
            v1 = [b_ref[n, ch] for ch in range(3)]
            o = 0
            for dy in range(-2, 3):
                for dx in range(-2, 3):
                    acc = None
                    for ch in range(3):
                        d = v0[ch] - _shift_edge(v1[ch], dy, dx)
                        acc = d * d if acc is None else acc + d * d
                    dref[n, o] = -acc
                    o += 1


def _search_body(tgt, qs, cyi, cxi, fqy, fqx, out,
                 t0, t1, t2, qb, cyb, cxb, fyb, fxb, ob):
    wid = lax.axis_index("s") * 2 + lax.axis_index("c")
    qr0 = wid * _QROWS
    rs = lax.min(lax.max(qr0 * 2 - 20, 0), _H - _HALO)

    nprob = out.shape[0]

    def do_problem(pi, carry):
        pltpu.sync_copy(tgt.at[pi, 0, pl.ds(rs, _HALO), :], t0)
        pltpu.sync_copy(tgt.at[pi, 1, pl.ds(rs, _HALO), :], t1)
        pltpu.sync_copy(tgt.at[pi, 2, pl.ds(rs, _HALO), :], t2)
        pltpu.sync_copy(qs.at[pi, :, pl.ds(qr0, _QROWS), :], qb)
        pltpu.sync_copy(cyi.at[pi, pl.ds(qr0, _QROWS), :], cyb)
        pltpu.sync_copy(cxi.at[pi, pl.ds(qr0, _QROWS), :], cxb)
        pltpu.sync_copy(fqy.at[pi, pl.ds(qr0, _QROWS), :], fyb)
        pltpu.sync_copy(fqx.at[pi, pl.ds(qr0, _QROWS), :], fxb)

        @plsc.parallel_loop(0, _QROWS * 8, unroll=2)
        def do_group(g):
            r = lax.shift_right_logical(g, 3)
            go = lax.mul(lax.rem(g, 8), 16)
            q0 = qb[0, r, pl.ds(go, 16)]
            q1 = qb[1, r, pl.ds(go, 16)]
            q2 = qb[2, r, pl.ds(go, 16)]
            cyv = cyb[r, pl.ds(go, 16)]
            cxv = cxb[r, pl.ds(go, 16)]
            fyv = fyb[r, pl.ds(go, 16)]
            fxv = fxb[r, pl.ds(go, 16)]

            rly = [jnp.minimum(jnp.maximum(cyv + dy, 0), _H - 1) - rs
                   for dy in range(-_WR, _WR + 1)]
            clx = [jnp.minimum(jnp.maximum(cxv + dx, 0), _H - 1)
                   for dx in range(-_WR, _WR + 1)]

            td = [jnp.full((16,), jnp.inf, jnp.float32)] * _K
            tj = [jnp.zeros((16,), jnp.int32)] * _K
            j = 0
            for iy in range(2 * _WR + 1):
                for ix in range(2 * _WR + 1):
                    v0 = plsc.load_gather(t0, [rly[iy], clx[ix]])
                    v1 = plsc.load_gather(t1, [rly[iy], clx[ix]])
                    v2 = plsc.load_gather(t2, [rly[iy], clx[ix]])
                    d0 = q0 - v0
                    d1 = q1 - v1
                    d2 = q2 - v2
                    dist = (d0 * d0 + d1 * d1) + d2 * d2
                    jv = jnp.full((16,), j, jnp.int32)
                    cs = [dist < td[i] for i in range(_K)]
                    ntd, ntj = [], []
                    for i in range(_K):
                        if i == 0:
                            nd = jnp.where(cs[0], dist, td[0])
                            nj = jnp.where(cs[0], jv, tj[0])
                        else:
                            sd = jnp.where(cs[i - 1], td[i - 1], dist)
                            sj = jnp.where(cs[i - 1], tj[i - 1], jv)
                            nd = jnp.where(cs[i], sd, td[i])
                            nj = jnp.where(cs[i], sj, tj[i])
                        ntd.append(nd)
                        ntj.append(nj)
                    td, tj = ntd, ntj
                    j += 1

            for k in range(_K):
                jdy = tj[k] // 9
                jdx = tj[k] - jdy * 9
                ob[2 * k, r, pl.ds(go, 16)] = fyv + (
                    jdy.astype(jnp.float32) - float(_WR))
                ob[2 * k + 1, r, pl.ds(go, 16)] = fxv + (
                    jdx.astype(jnp.float32) - float(_WR))

        for k in range(_K):
            for comp in range(2):
                pltpu.sync_copy(ob.at[2 * k + comp],
                                out.at[pi, k, comp, pl.ds(qr0, _QROWS), :])
        return carry

    lax.fori_loop(0, nprob, do_problem, 0)


_search_fn_cache = {}


def _get_search(nprob):
    if nprob not in _search_fn_cache:
        mesh = plsc.VectorSubcoreMesh(core_axis_name="c", subcore_axis_name="s")
        _search_fn_cache[nprob] = pl.kernel(
            _search_body,
            out_type=jax.ShapeDtypeStruct(
                (nprob, _K, 2, _NQ, _NQ), jnp.float32),
            mesh=mesh,
            compiler_params=pltpu.CompilerParams(
                use_tc_tiling_on_sc=False, needs_layout_passes=False),
            scratch_types=[
                pltpu.VMEM((_HALO, _H), jnp.float32),
                pltpu.VMEM((_HALO, _H), jnp.float32),
                pltpu.VMEM((_HALO, _H), jnp.float32),
                pltpu.VMEM((3, _QROWS, _NQ), jnp.float32),
                pltpu.VMEM((_QROWS, _NQ), jnp.int32),
                pltpu.VMEM((_QROWS, _NQ), jnp.int32),
                pltpu.VMEM((_QROWS, _NQ), jnp.float32),
                pltpu.VMEM((_QROWS, _NQ), jnp.float32),
                pltpu.VMEM((2 * _K, _QROWS, _NQ), jnp.float32),
            ],
        )
    return _search_fn_cache[nprob]


def _xla_pool(x, f):
    n, c, h, w = x.shape
    return x.reshape(n, c, h // f, f, w // f, f).mean(axis=(3, 5))


def kernel(vid):
    c0 = _xla_pool(vid[0, :-1], 8)
    c1 = _xla_pool(vid[0, 1:], 8)
    corr_f, corr_b = pl.pallas_call(
        _corr_body,
        out_shape=[
            jax.ShapeDtypeStruct((_NF, 25, _CG, _CG), jnp.float32),
            jax.ShapeDtypeStruct((_NF, 25, _CG, _CG), jnp.float32),
        ],
    )(c0, c1)

    offs = [(dy, dx) for dy in range(-2, 3) for dx in range(-2, 3)]
    off = jnp.asarray(offs, dtype=jnp.float32)
    gy = (jnp.arange(_NQ) * 2).astype(jnp.float32)
    gx = (jnp.arange(_NQ) * 2).astype(jnp.float32)
    cyis, cxis, fqys, fqxs = [], [], [], []
    for corr in (corr_f, corr_b):
        att = jax.nn.softmax(corr * 10.0, axis=1)
        flow = jnp.einsum('nohw,od->ndhw', att, off)
        flow = jnp.repeat(jnp.repeat(flow, 8, axis=2), 8, axis=3) * 8
        fq = flow[:, :, ::2, ::2]
        cy = gy[None, :, None] + fq[:, 0]
        cx = gx[None, None, :] + fq[:, 1]
        cyis.append(jnp.round(cy).astype(jnp.int32))
        cxis.append(jnp.round(cx).astype(jnp.int32))
        fqys.append(fq[:, 0])
        fqxs.append(fq[:, 1])
    cyi = jnp.stack(cyis)
    cxi = jnp.stack(cxis)
    fqy = jnp.stack(fqys)
    fqx = jnp.stack(fqxs)

    f0 = vid[0, :-1]
    f1 = vid[0, 1:]
    tgt = jnp.concatenate([f1, f0], axis=0)
    qs = jnp.concatenate(
        [f0[:, :, ::2, ::2], f1[:, :, ::2, ::2]], axis=0)

    cyi6 = cyi.reshape(_NP, _NQ, _NQ)
    cxi6 = cxi.reshape(_NP, _NQ, _NQ)
    fqy6 = fqy.reshape(_NP, _NQ, _NQ)
    fqx6 = fqx.reshape(_NP, _NQ, _NQ)
    search = _get_search(_NP // 2)
    out_a = search(tgt[:3], qs[:3], cyi6[:3], cxi6[:3], fqy6[:3], fqx6[:3])
    out_b = search(tgt[3:], qs[3:], cyi6[3:], cxi6[3:], fqy6[3:], fqx6[3:])
    out6 = jnp.concatenate([out_a, out_b], axis=0)
    arr = out6.reshape(2, _NF, _K, 2, _NQ, _NQ)
    arr = jnp.transpose(arr, (1, 0, 4, 5, 2, 3))
    return arr.reshape(1, _NF, 2, _NQ, _NQ, _K, 2)


# speedup vs baseline: 1.6384x; 1.6384x over previous
import functools

import jax
import jax.numpy as jnp
from jax import lax
from jax.experimental import pallas as pl
from jax.experimental.pallas import tpu as pltpu
from jax.experimental.pallas import tpu_sc as plsc

_H = 256
_CG = 32
_NQ = 128
_NF = 3
_NP = 6
_K = 8
_WR = 4
_QROWS = 4
_HALO = 48


def _shift_edge(a, dy, dx):
    if dy > 0:
        a = jnp.concatenate(
            [a[dy:, :], jnp.broadcast_to(a[_CG - 1:_CG, :], (dy, _CG))], axis=0)
    elif dy < 0:
        a = jnp.concatenate(
            [jnp.broadcast_to(a[0:1, :], (-dy, _CG)), a[:dy, :]], axis=0)
    if dx > 0:
        a = jnp.concatenate(
            [a[:, dx:], jnp.broadcast_to(a[:, _CG - 1:_CG], (_CG, dx))], axis=1)
    elif dx < 0:
        a = jnp.concatenate(
            [jnp.broadcast_to(a[:, 0:1], (_CG, -dx)), a[:, :dx]], axis=1)
    return a


def _corr_body(c0_ref, c1_ref, cf_ref, cb_ref):
    for n in range(_NF):
        for dref, (a_ref, b_ref) in ((cf_ref, (c0_ref, c1_ref)),
                                     (cb_ref, (c1_ref, c0_ref))):
            v0 = [a_ref[n, ch] for ch in range(3)]
            v1 = [b_ref[n, ch] for ch in range(3)]
            o = 0
            for dy in range(-2, 3):
                for dx in range(-2, 3):
                    acc = None
                    for ch in range(3):
                        d = v0[ch] - _shift_edge(v1[ch], dy, dx)
                        acc = d * d if acc is None else acc + d * d
                    dref[n, o] = -acc
                    o += 1


def _search_body(tgt, qs, cyi, cxi, fqy, fqx, out,
                 t0, t1, t2, qb, cyb, cxb, fyb, fxb, ob):
    wid = lax.axis_index("s") * 2 + lax.axis_index("c")
    qr0 = wid * _QROWS
    rs = lax.min(lax.max(qr0 * 2 - 20, 0), _H - _HALO)

    nprob = out.shape[0]

    def do_problem(pi, carry):
        pltpu.sync_copy(tgt.at[pi, 0, pl.ds(rs, _HALO), :], t0)
        pltpu.sync_copy(tgt.at[pi, 1, pl.ds(rs, _HALO), :], t1)
        pltpu.sync_copy(tgt.at[pi, 2, pl.ds(rs, _HALO), :], t2)
        pltpu.sync_copy(qs.at[pi, :, pl.ds(qr0, _QROWS), :], qb)
        pltpu.sync_copy(cyi.at[pi, pl.ds(qr0, _QROWS), :], cyb)
        pltpu.sync_copy(cxi.at[pi, pl.ds(qr0, _QROWS), :], cxb)
        pltpu.sync_copy(fqy.at[pi, pl.ds(qr0, _QROWS), :], fyb)
        pltpu.sync_copy(fqx.at[pi, pl.ds(qr0, _QROWS), :], fxb)

        @plsc.parallel_loop(0, _QROWS * 8, unroll=1)
        def do_group(g):
            r = lax.shift_right_logical(g, 3)
            go = lax.mul(lax.rem(g, 8), 16)
            q0 = qb[0, r, pl.ds(go, 16)]
            q1 = qb[1, r, pl.ds(go, 16)]
            q2 = qb[2, r, pl.ds(go, 16)]
            cyv = cyb[r, pl.ds(go, 16)]
            cxv = cxb[r, pl.ds(go, 16)]
            fyv = fyb[r, pl.ds(go, 16)]
            fxv = fxb[r, pl.ds(go, 16)]

            rly = [jnp.minimum(jnp.maximum(cyv + dy, 0), _H - 1) - rs
                   for dy in range(-_WR, _WR + 1)]
            clx = [jnp.minimum(jnp.maximum(cxv + dx, 0), _H - 1)
                   for dx in range(-_WR, _WR + 1)]

            td = [jnp.full((16,), jnp.inf, jnp.float32)] * _K
            tj = [jnp.zeros((16,), jnp.int32)] * _K
            j = 0
            for iy in range(2 * _WR + 1):
                for ix in range(2 * _WR + 1):
                    v0 = plsc.load_gather(t0, [rly[iy], clx[ix]])
                    v1 = plsc.load_gather(t1, [rly[iy], clx[ix]])
                    v2 = plsc.load_gather(t2, [rly[iy], clx[ix]])
                    d0 = q0 - v0
                    d1 = q1 - v1
                    d2 = q2 - v2
                    dist = (d0 * d0 + d1 * d1) + d2 * d2
                    jv = jnp.full((16,), j, jnp.int32)
                    cs = [dist < td[i] for i in range(_K)]
                    ntd, ntj = [], []
                    for i in range(_K):
                        if i == 0:
                            nd = jnp.where(cs[0], dist, td[0])
                            nj = jnp.where(cs[0], jv, tj[0])
                        else:
                            sd = jnp.where(cs[i - 1], td[i - 1], dist)
                            sj = jnp.where(cs[i - 1], tj[i - 1], jv)
                            nd = jnp.where(cs[i], sd, td[i])
                            nj = jnp.where(cs[i], sj, tj[i])
                        ntd.append(nd)
                        ntj.append(nj)
                    td, tj = ntd, ntj
                    j += 1

            for k in range(_K):
                jdy = tj[k] // 9
                jdx = tj[k] - jdy * 9
                ob[2 * k, r, pl.ds(go, 16)] = fyv + (
                    jdy.astype(jnp.float32) - float(_WR))
                ob[2 * k + 1, r, pl.ds(go, 16)] = fxv + (
                    jdx.astype(jnp.float32) - float(_WR))

        for k in range(_K):
            for comp in range(2):
                pltpu.sync_copy(ob.at[2 * k + comp],
                                out.at[pi, k, comp, pl.ds(qr0, _QROWS), :])
        return carry

    lax.fori_loop(0, nprob, do_problem, 0)


_search_fn_cache = {}


def _get_search(nprob):
    if nprob not in _search_fn_cache:
        mesh = plsc.VectorSubcoreMesh(core_axis_name="c", subcore_axis_name="s")
        _search_fn_cache[nprob] = pl.kernel(
            _search_body,
            out_type=jax.ShapeDtypeStruct(
                (nprob, _K, 2, _NQ, _NQ), jnp.float32),
            mesh=mesh,
            compiler_params=pltpu.CompilerParams(
                use_tc_tiling_on_sc=False, needs_layout_passes=False),
            scratch_types=[
                pltpu.VMEM((_HALO, _H), jnp.float32),
                pltpu.VMEM((_HALO, _H), jnp.float32),
                pltpu.VMEM((_HALO, _H), jnp.float32),
                pltpu.VMEM((3, _QROWS, _NQ), jnp.float32),
                pltpu.VMEM((_QROWS, _NQ), jnp.int32),
                pltpu.VMEM((_QROWS, _NQ), jnp.int32),
                pltpu.VMEM((_QROWS, _NQ), jnp.float32),
                pltpu.VMEM((_QROWS, _NQ), jnp.float32),
                pltpu.VMEM((2 * _K, _QROWS, _NQ), jnp.float32),
            ],
        )
    return _search_fn_cache[nprob]


def _xla_pool(x, f):
    n, c, h, w = x.shape
    return x.reshape(n, c, h // f, f, w // f, f).mean(axis=(3, 5))


def kernel(vid):
    c0 = _xla_pool(vid[0, :-1], 8)
    c1 = _xla_pool(vid[0, 1:], 8)
    corr_f, corr_b = pl.pallas_call(
        _corr_body,
        out_shape=[
            jax.ShapeDtypeStruct((_NF, 25, _CG, _CG), jnp.float32),
            jax.ShapeDtypeStruct((_NF, 25, _CG, _CG), jnp.float32),
        ],
    )(c0, c1)

    offs = [(dy, dx) for dy in range(-2, 3) for dx in range(-2, 3)]
    off = jnp.asarray(offs, dtype=jnp.float32)
    gy = (jnp.arange(_NQ) * 2).astype(jnp.float32)
    gx = (jnp.arange(_NQ) * 2).astype(jnp.float32)
    cyis, cxis, fqys, fqxs = [], [], [], []
    for corr in (corr_f, corr_b):
        att = jax.nn.softmax(corr * 10.0, axis=1)
        flow = jnp.einsum('nohw,od->ndhw', att, off)
        flow = jnp.repeat(jnp.repeat(flow, 8, axis=2), 8, axis=3) * 8
        fq = flow[:, :, ::2, ::2]
        cy = gy[None, :, None] + fq[:, 0]
        cx = gx[None, None, :] + fq[:, 1]
        cyis.append(jnp.round(cy).astype(jnp.int32))
        cxis.append(jnp.round(cx).astype(jnp.int32))
        fqys.append(fq[:, 0])
        fqxs.append(fq[:, 1])
    cyi = jnp.stack(cyis)
    cxi = jnp.stack(cxis)
    fqy = jnp.stack(fqys)
    fqx = jnp.stack(fqxs)

    f0 = vid[0, :-1]
    f1 = vid[0, 1:]
    tgt = jnp.concatenate([f1, f0], axis=0)
    qs = jnp.concatenate(
        [f0[:, :, ::2, ::2], f1[:, :, ::2, ::2]], axis=0)

    cyi6 = cyi.reshape(_NP, _NQ, _NQ)
    cxi6 = cxi.reshape(_NP, _NQ, _NQ)
    fqy6 = fqy.reshape(_NP, _NQ, _NQ)
    fqx6 = fqx.reshape(_NP, _NQ, _NQ)
    search = _get_search(_NP // 2)
    out_a = search(tgt[:3], qs[:3], cyi6[:3], cxi6[:3], fqy6[:3], fqx6[:3])
    out_b = search(tgt[3:], qs[3:], cyi6[3:], cxi6[3:], fqy6[3:], fqx6[3:])
    out6 = jnp.concatenate([out_a, out_b], axis=0)
    arr = out6.reshape(2, _NF, _K, 2, _NQ, _NQ)
    arr = jnp.transpose(arr, (1, 0, 4, 5, 2, 3))
    return arr.reshape(1, _NF, 2, _NQ, _NQ, _K, 2)


# speedup vs baseline: 1.7403x; 1.0622x over previous
import functools

import jax
import jax.numpy as jnp
from jax import lax
from jax.experimental import pallas as pl
from jax.experimental.pallas import tpu as pltpu
from jax.experimental.pallas import tpu_sc as plsc

_H = 256
_CG = 32
_NQ = 128
_NF = 3
_NP = 6
_K = 8
_WR = 4
_QROWS = 4
_HALO = 48


def _shift_edge(a, dy, dx):
    if dy > 0:
        a = jnp.concatenate(
            [a[dy:, :], jnp.broadcast_to(a[_CG - 1:_CG, :], (dy, _CG))], axis=0)
    elif dy < 0:
        a = jnp.concatenate(
            [jnp.broadcast_to(a[0:1, :], (-dy, _CG)), a[:dy, :]], axis=0)
    if dx > 0:
        a = jnp.concatenate(
            [a[:, dx:], jnp.broadcast_to(a[:, _CG - 1:_CG], (_CG, dx))], axis=1)
    elif dx < 0:
        a = jnp.concatenate(
            [jnp.broadcast_to(a[:, 0:1], (_CG, -dx)), a[:, :dx]], axis=1)
    return a


def _corr_body(c0_ref, c1_ref, cf_ref, cb_ref):
    for n in range(_NF):
        for dref, (a_ref, b_ref) in ((cf_ref, (c0_ref, c1_ref)),
                                     (cb_ref, (c1_ref, c0_ref))):
            v0 = [a_ref[n, ch] for ch in range(3)]
            v1 = [b_ref[n, ch] for ch in range(3)]
            o = 0
            for dy in range(-2, 3):
                for dx in range(-2, 3):
                    acc = None
                    for ch in range(3):
                        d = v0[ch] - _shift_edge(v1[ch], dy, dx)
                        acc = d * d if acc is None else acc + d * d
                    dref[n, o] = -acc
                    o += 1


def _search_body(tgt, qs, cyi, cxi, fqy, fqx, out,
                 t0, t1, t2, qb, cyb, cxb, fyb, fxb, ob, insem):
    wid = lax.axis_index("s") * 2 + lax.axis_index("c")
    qr0 = wid * _QROWS
    rs = lax.min(lax.max(qr0 * 2 - 20, 0), _H - _HALO)

    nprob = out.shape[0]

    def do_problem(pi, carry):
        cps = [
            pltpu.async_copy(tgt.at[pi, 0, pl.ds(rs, _HALO), :], t0, insem),
            pltpu.async_copy(tgt.at[pi, 1, pl.ds(rs, _HALO), :], t1, insem),
            pltpu.async_copy(tgt.at[pi, 2, pl.ds(rs, _HALO), :], t2, insem),
            pltpu.async_copy(qs.at[pi, :, pl.ds(qr0, _QROWS), :], qb, insem),
            pltpu.async_copy(cyi.at[pi, pl.ds(qr0, _QROWS), :], cyb, insem),
            pltpu.async_copy(cxi.at[pi, pl.ds(qr0, _QROWS), :], cxb, insem),
            pltpu.async_copy(fqy.at[pi, pl.ds(qr0, _QROWS), :], fyb, insem),
            pltpu.async_copy(fqx.at[pi, pl.ds(qr0, _QROWS), :], fxb, insem),
        ]
        for cp in cps:
            cp.wait()

        @plsc.parallel_loop(0, _QROWS * 8, unroll=1)
        def do_group(g):
            r = lax.shift_right_logical(g, 3)
            go = lax.mul(lax.rem(g, 8), 16)
            q0 = qb[0, r, pl.ds(go, 16)]
            q1 = qb[1, r, pl.ds(go, 16)]
            q2 = qb[2, r, pl.ds(go, 16)]
            cyv = cyb[r, pl.ds(go, 16)]
            cxv = cxb[r, pl.ds(go, 16)]
            fyv = fyb[r, pl.ds(go, 16)]
            fxv = fxb[r, pl.ds(go, 16)]

            rly = [jnp.minimum(jnp.maximum(cyv + dy, 0), _H - 1) - rs
                   for dy in range(-_WR, _WR + 1)]
            clx = [jnp.minimum(jnp.maximum(cxv + dx, 0), _H - 1)
                   for dx in range(-_WR, _WR + 1)]

            td = [jnp.full((16,), jnp.inf, jnp.float32)] * _K
            tj = [jnp.zeros((16,), jnp.int32)] * _K
            j = 0
            for iy in range(2 * _WR + 1):
                for ix in range(2 * _WR + 1):
                    v0 = plsc.load_gather(t0, [rly[iy], clx[ix]])
                    v1 = plsc.load_gather(t1, [rly[iy], clx[ix]])
                    v2 = plsc.load_gather(t2, [rly[iy], clx[ix]])
                    d0 = q0 - v0
                    d1 = q1 - v1
                    d2 = q2 - v2
                    dist = (d0 * d0 + d1 * d1) + d2 * d2
                    jv = jnp.full((16,), j, jnp.int32)
                    cs = [dist < td[i] for i in range(_K)]
                    ntd, ntj = [], []
                    for i in range(_K):
                        if i == 0:
                            nd = jnp.where(cs[0], dist, td[0])
                            nj = jnp.where(cs[0], jv, tj[0])
                        else:
                            sd = jnp.where(cs[i - 1], td[i - 1], dist)
                            sj = jnp.where(cs[i - 1], tj[i - 1], jv)
                            nd = jnp.where(cs[i], sd, td[i])
                            nj = jnp.where(cs[i], sj, tj[i])
                        ntd.append(nd)
                        ntj.append(nj)
                    td, tj = ntd, ntj
                    j += 1

            for k in range(_K):
                jdy = tj[k] // 9
                jdx = tj[k] - jdy * 9
                ob[2 * k, r, pl.ds(go, 16)] = fyv + (
                    jdy.astype(jnp.float32) - float(_WR))
                ob[2 * k + 1, r, pl.ds(go, 16)] = fxv + (
                    jdx.astype(jnp.float32) - float(_WR))

        for k in range(_K):
            for comp in range(2):
                pltpu.sync_copy(ob.at[2 * k + comp],
                                out.at[pi, k, comp, pl.ds(qr0, _QROWS), :])
        return carry

    lax.fori_loop(0, nprob, do_problem, 0)


_search_fn_cache = {}


def _get_search(nprob):
    if nprob not in _search_fn_cache:
        mesh = plsc.VectorSubcoreMesh(core_axis_name="c", subcore_axis_name="s")
        _search_fn_cache[nprob] = pl.kernel(
            _search_body,
            out_type=jax.ShapeDtypeStruct(
                (nprob, _K, 2, _NQ, _NQ), jnp.float32),
            mesh=mesh,
            compiler_params=pltpu.CompilerParams(
                use_tc_tiling_on_sc=False, needs_layout_passes=False),
            scratch_types=[
                pltpu.VMEM((_HALO, _H), jnp.float32),
                pltpu.VMEM((_HALO, _H), jnp.float32),
                pltpu.VMEM((_HALO, _H), jnp.float32),
                pltpu.VMEM((3, _QROWS, _NQ), jnp.float32),
                pltpu.VMEM((_QROWS, _NQ), jnp.int32),
                pltpu.VMEM((_QROWS, _NQ), jnp.int32),
                pltpu.VMEM((_QROWS, _NQ), jnp.float32),
                pltpu.VMEM((_QROWS, _NQ), jnp.float32),
                pltpu.VMEM((2 * _K, _QROWS, _NQ), jnp.float32),
                pltpu.SemaphoreType.DMA,
            ],
        )
    return _search_fn_cache[nprob]


def _xla_pool(x, f):
    n, c, h, w = x.shape
    return x.reshape(n, c, h // f, f, w // f, f).mean(axis=(3, 5))


def kernel(vid):
    c0 = _xla_pool(vid[0, :-1], 8)
    c1 = _xla_pool(vid[0, 1:], 8)
    corr_f, corr_b = pl.pallas_call(
        _corr_body,
        out_shape=[
            jax.ShapeDtypeStruct((_NF, 25, _CG, _CG), jnp.float32),
            jax.ShapeDtypeStruct((_NF, 25, _CG, _CG), jnp.float32),
        ],
    )(c0, c1)

    offs = [(dy, dx) for dy in range(-2, 3) for dx in range(-2, 3)]
    off = jnp.asarray(offs, dtype=jnp.float32)
    gy = (jnp.arange(_NQ) * 2).astype(jnp.float32)
    gx = (jnp.arange(_NQ) * 2).astype(jnp.float32)
    cyis, cxis, fqys, fqxs = [], [], [], []
    for corr in (corr_f, corr_b):
        att = jax.nn.softmax(corr * 10.0, axis=1)
        flow = jnp.einsum('nohw,od->ndhw', att, off)
        flow = jnp.repeat(jnp.repeat(flow, 8, axis=2), 8, axis=3) * 8
        fq = flow[:, :, ::2, ::2]
        cy = gy[None, :, None] + fq[:, 0]
        cx = gx[None, None, :] + fq[:, 1]
        cyis.append(jnp.round(cy).astype(jnp.int32))
        cxis.append(jnp.round(cx).astype(jnp.int32))
        fqys.append(fq[:, 0])
        fqxs.append(fq[:, 1])
    cyi = jnp.stack(cyis)
    cxi = jnp.stack(cxis)
    fqy = jnp.stack(fqys)
    fqx = jnp.stack(fqxs)

    f0 = vid[0, :-1]
    f1 = vid[0, 1:]
    tgt = jnp.concatenate([f1, f0], axis=0)
    qs = jnp.concatenate(
        [f0[:, :, ::2, ::2], f1[:, :, ::2, ::2]], axis=0)

    cyi6 = cyi.reshape(_NP, _NQ, _NQ)
    cxi6 = cxi.reshape(_NP, _NQ, _NQ)
    fqy6 = fqy.reshape(_NP, _NQ, _NQ)
    fqx6 = fqx.reshape(_NP, _NQ, _NQ)
    search = _get_search(_NP // 2)
    out_a = search(tgt[:3], qs[:3], cyi6[:3], cxi6[:3], fqy6[:3], fqx6[:3])
    out_b = search(tgt[3:], qs[3:], cyi6[3:], cxi6[3:], fqy6[3:], fqx6[3:])
    out6 = jnp.concatenate([out_a, out_b], axis=0)
    arr = out6.reshape(2, _NF, _K, 2, _NQ, _NQ)
    arr = jnp.transpose(arr, (1, 0, 4, 5, 2, 3))
    return arr.reshape(1, _NF, 2, _NQ, _NQ, _K, 2)


# speedup vs baseline: 1.7730x; 1.0188x over previous
import functools

import jax
import jax.numpy as jnp
from jax import lax
from jax.experimental import pallas as pl
from jax.experimental.pallas import tpu as pltpu
from jax.experimental.pallas import tpu_sc as plsc

_H = 256
_CG = 32
_NQ = 128
_NF = 3
_NP = 6
_K = 8
_WR = 4
_QROWS = 4
_HALO = 48


def _shift_edge(a, dy, dx):
    if dy > 0:
        a = jnp.concatenate(
            [a[dy:, :], jnp.broadcast_to(a[_CG - 1:_CG, :], (dy, _CG))], axis=0)
    elif dy < 0:
        a = jnp.concatenate(
            [jnp.broadcast_to(a[0:1, :], (-dy, _CG)), a[:dy, :]], axis=0)
    if dx > 0:
        a = jnp.concatenate(
            [a[:, dx:], jnp.broadcast_to(a[:, _CG - 1:_CG], (_CG, dx))], axis=1)
    elif dx < 0:
        a = jnp.concatenate(
            [jnp.broadcast_to(a[:, 0:1], (_CG, -dx)), a[:, :dx]], axis=1)
    return a


def _corr_body(c0_ref, c1_ref, cf_ref, cb_ref):
    for n in range(_NF):
        for dref, (a_ref, b_ref) in ((cf_ref, (c0_ref, c1_ref)),
                                     (cb_ref, (c1_ref, c0_ref))):
            v0 = [a_ref[n, ch] for ch in range(3)]
            v1 = [b_ref[n, ch] for ch in range(3)]
            o = 0
            for dy in range(-2, 3):
                for dx in range(-2, 3):
                    acc = None
                    for ch in range(3):
                        d = v0[ch] - _shift_edge(v1[ch], dy, dx)
                        acc = d * d if acc is None else acc + d * d
                    dref[n, o] = -acc
                    o += 1


def _search_body(tgt, qs, cyi, cxi, fqy, fqx, out,
                 t0, t1, t2, qb, cyb, cxb, fyb, fxb, ob, insem, outsem):
    wid = lax.axis_index("s") * 2 + lax.axis_index("c")
    qr0 = wid * _QROWS
    rs = lax.min(lax.max(qr0 * 2 - 20, 0), _H - _HALO)

    nprob = out.shape[0]

    def do_problem(pi, carry):
        cps = [
            pltpu.async_copy(tgt.at[pi, 0, pl.ds(rs, _HALO), :], t0, insem),
            pltpu.async_copy(tgt.at[pi, 1, pl.ds(rs, _HALO), :], t1, insem),
            pltpu.async_copy(tgt.at[pi, 2, pl.ds(rs, _HALO), :], t2, insem),
            pltpu.async_copy(qs.at[pi, :, pl.ds(qr0, _QROWS), :], qb, insem),
            pltpu.async_copy(cyi.at[pi, pl.ds(qr0, _QROWS), :], cyb, insem),
            pltpu.async_copy(cxi.at[pi, pl.ds(qr0, _QROWS), :], cxb, insem),
            pltpu.async_copy(fqy.at[pi, pl.ds(qr0, _QROWS), :], fyb, insem),
            pltpu.async_copy(fqx.at[pi, pl.ds(qr0, _QROWS), :], fxb, insem),
        ]
        for cp in cps:
            cp.wait()

        @pl.when(pi > 0)
        def _drain_prev_out():
            for k in range(_K):
                for comp in range(2):
                    pltpu.make_async_copy(
                        ob.at[2 * k + comp],
                        out.at[pi, k, comp, pl.ds(qr0, _QROWS), :],
                        outsem).wait()

        @plsc.parallel_loop(0, _QROWS * 8, unroll=1)
        def do_group(g):
            r = lax.shift_right_logical(g, 3)
            go = lax.mul(lax.rem(g, 8), 16)
            q0 = qb[0, r, pl.ds(go, 16)]
            q1 = qb[1, r, pl.ds(go, 16)]
            q2 = qb[2, r, pl.ds(go, 16)]
            cyv = cyb[r, pl.ds(go, 16)]
            cxv = cxb[r, pl.ds(go, 16)]
            fyv = fyb[r, pl.ds(go, 16)]
            fxv = fxb[r, pl.ds(go, 16)]

            rly = [jnp.minimum(jnp.maximum(cyv + dy, 0), _H - 1) - rs
                   for dy in range(-_WR, _WR + 1)]
            clx = [jnp.minimum(jnp.maximum(cxv + dx, 0), _H - 1)
                   for dx in range(-_WR, _WR + 1)]

            td = [jnp.full((16,), jnp.inf, jnp.float32)] * _K
            tj = [jnp.zeros((16,), jnp.int32)] * _K
            j = 0
            for iy in range(2 * _WR + 1):
                for ix in range(2 * _WR + 1):
                    v0 = plsc.load_gather(t0, [rly[iy], clx[ix]])
                    v1 = plsc.load_gather(t1, [rly[iy], clx[ix]])
                    v2 = plsc.load_gather(t2, [rly[iy], clx[ix]])
                    d0 = q0 - v0
                    d1 = q1 - v1
                    d2 = q2 - v2
                    dist = (d0 * d0 + d1 * d1) + d2 * d2
                    jv = jnp.full((16,), j, jnp.int32)
                    cs = [dist < td[i] for i in range(_K)]
                    ntd, ntj = [], []
                    for i in range(_K):
                        if i == 0:
                            nd = jnp.where(cs[0], dist, td[0])
                            nj = jnp.where(cs[0], jv, tj[0])
                        else:
                            sd = jnp.where(cs[i - 1], td[i - 1], dist)
                            sj = jnp.where(cs[i - 1], tj[i - 1], jv)
                            nd = jnp.where(cs[i], sd, td[i])
                            nj = jnp.where(cs[i], sj, tj[i])
                        ntd.append(nd)
                        ntj.append(nj)
                    td, tj = ntd, ntj
                    j += 1

            for k in range(_K):
                jdy = tj[k] // 9
                jdx = tj[k] - jdy * 9
                ob[2 * k, r, pl.ds(go, 16)] = fyv + (
                    jdy.astype(jnp.float32) - float(_WR))
                ob[2 * k + 1, r, pl.ds(go, 16)] = fxv + (
                    jdx.astype(jnp.float32) - float(_WR))

        for k in range(_K):
            for comp in range(2):
                pltpu.async_copy(ob.at[2 * k + comp],
                                 out.at[pi, k, comp, pl.ds(qr0, _QROWS), :],
                                 outsem)
        return carry

    lax.fori_loop(0, nprob, do_problem, 0)
    for k in range(_K):
        for comp in range(2):
            pltpu.make_async_copy(
                ob.at[2 * k + comp],
                out.at[nprob - 1, k, comp, pl.ds(qr0, _QROWS), :],
                outsem).wait()


_search_fn_cache = {}


def _get_search(nprob):
    if nprob not in _search_fn_cache:
        mesh = plsc.VectorSubcoreMesh(core_axis_name="c", subcore_axis_name="s")
        _search_fn_cache[nprob] = pl.kernel(
            _search_body,
            out_type=jax.ShapeDtypeStruct(
                (nprob, _K, 2, _NQ, _NQ), jnp.float32),
            mesh=mesh,
            compiler_params=pltpu.CompilerParams(
                use_tc_tiling_on_sc=False, needs_layout_passes=False),
            scratch_types=[
                pltpu.VMEM((_HALO, _H), jnp.float32),
                pltpu.VMEM((_HALO, _H), jnp.float32),
                pltpu.VMEM((_HALO, _H), jnp.float32),
                pltpu.VMEM((3, _QROWS, _NQ), jnp.float32),
                pltpu.VMEM((_QROWS, _NQ), jnp.int32),
                pltpu.VMEM((_QROWS, _NQ), jnp.int32),
                pltpu.VMEM((_QROWS, _NQ), jnp.float32),
                pltpu.VMEM((_QROWS, _NQ), jnp.float32),
                pltpu.VMEM((2 * _K, _QROWS, _NQ), jnp.float32),
                pltpu.SemaphoreType.DMA,
                pltpu.SemaphoreType.DMA,
            ],
        )
    return _search_fn_cache[nprob]


def _xla_pool(x, f):
    n, c, h, w = x.shape
    return x.reshape(n, c, h // f, f, w // f, f).mean(axis=(3, 5))


def kernel(vid):
    c0 = _xla_pool(vid[0, :-1], 8)
    c1 = _xla_pool(vid[0, 1:], 8)
    corr_f, corr_b = pl.pallas_call(
        _corr_body,
        out_shape=[
            jax.ShapeDtypeStruct((_NF, 25, _CG, _CG), jnp.float32),
            jax.ShapeDtypeStruct((_NF, 25, _CG, _CG), jnp.float32),
        ],
    )(c0, c1)

    offs = [(dy, dx) for dy in range(-2, 3) for dx in range(-2, 3)]
    off = jnp.asarray(offs, dtype=jnp.float32)
    gy = (jnp.arange(_NQ) * 2).astype(jnp.float32)
    gx = (jnp.arange(_NQ) * 2).astype(jnp.float32)
    cyis, cxis, fqys, fqxs = [], [], [], []
    for corr in (corr_f, corr_b):
        att = jax.nn.softmax(corr * 10.0, axis=1)
        flow = jnp.einsum('nohw,od->ndhw', att, off)
        flow = jnp.repeat(jnp.repeat(flow, 8, axis=2), 8, axis=3) * 8
        fq = flow[:, :, ::2, ::2]
        cy = gy[None, :, None] + fq[:, 0]
        cx = gx[None, None, :] + fq[:, 1]
        cyis.append(jnp.round(cy).astype(jnp.int32))
        cxis.append(jnp.round(cx).astype(jnp.int32))
        fqys.append(fq[:, 0])
        fqxs.append(fq[:, 1])
    cyi = jnp.stack(cyis)
    cxi = jnp.stack(cxis)
    fqy = jnp.stack(fqys)
    fqx = jnp.stack(fqxs)

    f0 = vid[0, :-1]
    f1 = vid[0, 1:]
    tgt = jnp.concatenate([f1, f0], axis=0)
    qs = jnp.concatenate(
        [f0[:, :, ::2, ::2], f1[:, :, ::2, ::2]], axis=0)

    cyi6 = cyi.reshape(_NP, _NQ, _NQ)
    cxi6 = cxi.reshape(_NP, _NQ, _NQ)
    fqy6 = fqy.reshape(_NP, _NQ, _NQ)
    fqx6 = fqx.reshape(_NP, _NQ, _NQ)
    search = _get_search(_NP // 2)
    out_a = search(tgt[:3], qs[:3], cyi6[:3], cxi6[:3], fqy6[:3], fqx6[:3])
    out_b = search(tgt[3:], qs[3:], cyi6[3:], cxi6[3:], fqy6[3:], fqx6[3:])
    out6 = jnp.concatenate([out_a, out_b], axis=0)
    arr = out6.reshape(2, _NF, _K, 2, _NQ, _NQ)
    arr = jnp.transpose(arr, (1, 0, 4, 5, 2, 3))
    return arr.reshape(1, _NF, 2, _NQ, _NQ, _K, 2)
